# baseline (speedup 1.0000x reference)
import functools

import jax
import jax.numpy as jnp
from jax import lax
from jax.experimental import pallas as pl
from jax.experimental.pallas import tpu as pltpu
from jax.experimental.pallas import tpu_sc as plsc

N = 10000
NPAD = 10240
E = 320000
D = 128
F = 16

NC = 2
NS = 16
NW = NC * NS
CHUNK = 128
SUP = 4
NSUPER = 20
NCHUNK = SUP * NSUPER
EPT = NCHUNK * CHUNK
E_PAD = NW * EPT
NSALLOC = NSUPER + 2
ROWS_PT = NPAD // NS

_mesh = plsc.VectorSubcoreMesh(core_axis_name="c", subcore_axis_name="s")
_sc_params = pltpu.CompilerParams(use_tc_tiling_on_sc=False)


@functools.partial(
    pl.kernel,
    out_type=jax.ShapeDtypeStruct((NC, NPAD), jnp.float32),
    mesh=_mesh,
    scratch_types=[
        pltpu.VMEM_SHARED((NPAD,), jnp.float32),
        pltpu.VMEM((SUP, 2, CHUNK), jnp.int32),
        pltpu.VMEM((SUP, 2, CHUNK), jnp.int32),
        pltpu.VMEM((CHUNK,), jnp.float32),
        pltpu.VMEM((ROWS_PT,), jnp.float32),
        pltpu.SemaphoreType.DMA,
        pltpu.SemaphoreType.DMA,
        pltpu.SemaphoreType.DMA,
        pltpu.SemaphoreType.DMA,
    ],
    compiler_params=_sc_params,
)
def _sc_degree(eidx_hbm, out_hbm, acc, ib0, ib1, ones, stage, si0, si1, ss0, ss1):
    c = lax.axis_index("c")
    s = lax.axis_index("s")
    w = c * NS + s
    ibs, sis, sss = (ib0, ib1), (si0, si1), (ss0, ss1)

    def _fill(i, _):
        stage[pl.ds(i * 16, 16)] = jnp.zeros((16,), jnp.float32)
        return 0
    lax.fori_loop(0, ROWS_PT // 16, _fill, 0)

    def _fill1(i, _):
        ones[pl.ds(i * 16, 16)] = jnp.ones((16,), jnp.float32)
        return 0
    lax.fori_loop(0, CHUNK // 16, _fill1, 0)

    pltpu.sync_copy(stage, acc.at[pl.ds(s * ROWS_PT, ROWS_PT)])
    plsc.subcore_barrier()

    pltpu.async_copy(eidx_hbm.at[w, pl.ds(0, SUP)], ib0, si0)
    pltpu.async_copy(eidx_hbm.at[w, pl.ds(SUP, SUP)], ib1, si1)

    def _pair(i, _):
        for b in (0, 1):
            sk = 2 * i + b
            ib, si, ss = ibs[b], sis[b], sss[b]
            pltpu.make_async_copy(eidx_hbm.at[w, pl.ds(0, SUP)], ib, si).wait()
            for j in range(SUP):
                pltpu.async_copy(ones, acc.at[ib.at[j, 1]], ss, add=True)
            for j in range(SUP):
                pltpu.make_async_copy(ones, acc.at[ib.at[j, 1]], ss).wait()
            pltpu.async_copy(eidx_hbm.at[w, pl.ds((sk + 2) * SUP, SUP)], ib, si)
        return 0
    lax.fori_loop(0, NSUPER // 2, _pair, 0)
    pltpu.make_async_copy(eidx_hbm.at[w, pl.ds(0, SUP)], ib0, si0).wait()
    pltpu.make_async_copy(eidx_hbm.at[w, pl.ds(0, SUP)], ib1, si1).wait()

    plsc.subcore_barrier()
    pltpu.sync_copy(acc.at[pl.ds(s * ROWS_PT, ROWS_PT)], stage)
    pltpu.sync_copy(stage, out_hbm.at[c, pl.ds(s * ROWS_PT, ROWS_PT)])


@functools.partial(
    pl.kernel,
    out_type=jax.ShapeDtypeStruct((NC, NPAD, F), jnp.float32),
    mesh=_mesh,
    scratch_types=[
        pltpu.VMEM_SHARED((NPAD, F), jnp.float32),
        pltpu.VMEM((SUP, 2, CHUNK), jnp.int32),
        pltpu.VMEM((SUP, 2, CHUNK), jnp.int32),
        pltpu.VMEM((SUP, CHUNK, F), jnp.float32),
        pltpu.VMEM((SUP, CHUNK, F), jnp.float32),
        pltpu.SemaphoreType.DMA,
        pltpu.SemaphoreType.DMA,
        pltpu.SemaphoreType.DMA,
        pltpu.SemaphoreType.DMA,
        pltpu.SemaphoreType.DMA,
        pltpu.SemaphoreType.DMA,
    ],
    compiler_params=_sc_params,
)
def _sc_edge_pass(y_hbm, eidx_hbm, out_hbm,
                  acc, ib0, ib1, r0, r1, si0, si1, sg0, sg1, ss0, ss1):
    c = lax.axis_index("c")
    s = lax.axis_index("s")
    w = c * NS + s
    ibs, rs = (ib0, ib1), (r0, r1)
    sis, sgs, sss = (si0, si1), (sg0, sg1), (ss0, ss1)

    def _fill(i, _):
        r0[0, i, :] = jnp.zeros((16,), jnp.float32)
        return 0
    lax.fori_loop(0, CHUNK, _fill, 0)

    def _zinit(j, _):
        pltpu.sync_copy(r0.at[0], acc.at[pl.ds(s * ROWS_PT + j * CHUNK, CHUNK)])
        return 0
    lax.fori_loop(0, ROWS_PT // CHUNK, _zinit, 0)
    plsc.subcore_barrier()

    pltpu.async_copy(eidx_hbm.at[w, pl.ds(0, SUP)], ib0, si0)
    pltpu.async_copy(eidx_hbm.at[w, pl.ds(SUP, SUP)], ib1, si1)
    pltpu.make_async_copy(eidx_hbm.at[w, pl.ds(0, SUP)], ib0, si0).wait()
    for j in range(SUP):
        pltpu.async_copy(y_hbm.at[ib0.at[j, 0]], r0.at[j], sg0)

    def _pair(i, _):
        for b in (0, 1):
            sk = 2 * i + b
            b1 = 1 - b
            pltpu.make_async_copy(
                eidx_hbm.at[w, pl.ds(0, SUP)], ibs[b1], sis[b1]).wait()
            for j in range(SUP):
                pltpu.async_copy(y_hbm.at[ibs[b1].at[j, 0]], rs[b1].at[j], sgs[b1])
            for j in range(SUP):
                pltpu.make_async_copy(
                    y_hbm.at[ibs[b].at[j, 0]], rs[b].at[j], sgs[b]).wait()
            for j in range(SUP):
                pltpu.async_copy(rs[b].at[j], acc.at[ibs[b].at[j, 1]],
                                 sss[b], add=True)
            for j in range(SUP):
                pltpu.make_async_copy(
                    rs[b].at[j], acc.at[ibs[b].at[j, 1]], sss[b]).wait()
            pltpu.async_copy(eidx_hbm.at[w, pl.ds((sk + 2) * SUP, SUP)],
                             ibs[b], sis[b])
        return 0
    lax.fori_loop(0, NSUPER // 2, _pair, 0)
    for j in range(SUP):
        pltpu.make_async_copy(y_hbm.at[ib0.at[j, 0]], r0.at[j], sg0).wait()
    pltpu.make_async_copy(eidx_hbm.at[w, pl.ds(0, SUP)], ib1, si1).wait()

    plsc.subcore_barrier()

    def _copyout(j, _):
        sl = pl.ds(s * ROWS_PT + j * CHUNK, CHUNK)
        pltpu.sync_copy(acc.at[sl], r0.at[0])
        pltpu.sync_copy(r0.at[0], out_hbm.at[c, sl])
        return 0
    lax.fori_loop(0, ROWS_PT // CHUNK, _copyout, 0)


BLK = 1024


def _tc_prep_body(x_ref, w_ref, d0_ref, d1_ref, y_ref):
    dinv = lax.rsqrt(d0_ref[...] + d1_ref[...] + 1.0)
    xw = jnp.dot(x_ref[...], w_ref[...], preferred_element_type=jnp.float32)
    y_ref[...] = xw * dinv


def _tc_mid_body(p0_ref, p1_ref, y1_ref, d0_ref, d1_ref, b_ref, w_ref, y2_ref):
    i = pl.program_id(0)
    dinv = lax.rsqrt(d0_ref[...] + d1_ref[...] + 1.0)
    h = jnp.maximum(dinv * (p0_ref[...] + p1_ref[...] + y1_ref[...]) + b_ref[...], 0.0)
    xw2 = jnp.dot(h, w_ref[...], preferred_element_type=jnp.float32)
    row = lax.broadcasted_iota(jnp.int32, (BLK, F), 0) + i * BLK
    y2_ref[...] = jnp.where(row < N, xw2 * dinv, 0.0)


FBLK = 1000


def _tc_final_body(p0_ref, p1_ref, y2_ref, d0_ref, d1_ref, b_ref, o_ref):
    dinv = lax.rsqrt(d0_ref[...] + d1_ref[...] + 1.0)
    o_ref[...] = jnp.maximum(
        dinv * (p0_ref[...] + p1_ref[...] + y2_ref[...]) + b_ref[...], 0.0)


def _row_spec(blk, width):
    return pl.BlockSpec((blk, width), lambda i: (i, 0))


def _full_spec(shape):
    return pl.BlockSpec(shape, lambda i: (0, 0))


def kernel(x, edge_index, W1, b1, W2, b2):
    ei = edge_index.astype(jnp.int32)
    pad = jnp.full((E_PAD - E,), N, jnp.int32)
    srcr = jnp.concatenate([ei[0], pad]).reshape(NW, NCHUNK, CHUNK)
    dstr = jnp.concatenate([ei[1], pad]).reshape(NW, NCHUNK, CHUNK)
    eidx = jnp.pad(jnp.stack([srcr, dstr], axis=2),
                   ((0, 0), (0, (NSALLOC - NSUPER) * SUP), (0, 0), (0, 0)),
                   constant_values=N)
    xp = jnp.pad(x, ((0, NPAD - N), (0, 0)))
    b1r = b1.reshape(1, F)
    b2r = b2.reshape(1, F)

    deg_parts = _sc_degree(eidx)
    d0 = deg_parts[0].reshape(NPAD, 1)
    d1 = deg_parts[1].reshape(NPAD, 1)

    y1 = pl.pallas_call(
        _tc_prep_body,
        grid=(NPAD // BLK,),
        in_specs=[_row_spec(BLK, D), _full_spec((D, F)),
                  _row_spec(BLK, 1), _row_spec(BLK, 1)],
        out_specs=_row_spec(BLK, F),
        out_shape=jax.ShapeDtypeStruct((NPAD, F), jnp.float32),
    )(xp, W1, d0, d1)

    p = _sc_edge_pass(y1, eidx)

    y2 = pl.pallas_call(
        _tc_mid_body,
        grid=(NPAD // BLK,),
        in_specs=[_row_spec(BLK, F), _row_spec(BLK, F), _row_spec(BLK, F),
                  _row_spec(BLK, 1), _row_spec(BLK, 1),
                  _full_spec((1, F)), _full_spec((F, F))],
        out_specs=_row_spec(BLK, F),
        out_shape=jax.ShapeDtypeStruct((NPAD, F), jnp.float32),
    )(p[0], p[1], y1, d0, d1, b1r, W2)

    q = _sc_edge_pass(y2, eidx)

    out = pl.pallas_call(
        _tc_final_body,
        grid=(N // FBLK,),
        in_specs=[_row_spec(FBLK, F), _row_spec(FBLK, F), _row_spec(FBLK, F),
                  _row_spec(FBLK, 1), _row_spec(FBLK, 1), _full_spec((1, F))],
        out_specs=_row_spec(FBLK, F),
        out_shape=jax.ShapeDtypeStruct((N, F), jnp.float32),
    )(q[0], q[1], y2, d0, d1, b2r)

    return out


# speedup vs baseline: 1.0412x; 1.0412x over previous
import functools

import jax
import jax.numpy as jnp
from jax import lax
from jax.experimental import pallas as pl
from jax.experimental.pallas import tpu as pltpu
from jax.experimental.pallas import tpu_sc as plsc

N = 10000
NPAD = 10240
E = 320000
D = 128
F = 16

NC = 2
NS = 16
NW = NC * NS
CHUNK = 128
NSLOT = 4
NCHUNK = 80
EPT = NCHUNK * CHUNK
E_PAD = NW * EPT
NALLOC = NCHUNK + NSLOT
ROWS_PT = NPAD // NS

_mesh = plsc.VectorSubcoreMesh(core_axis_name="c", subcore_axis_name="s")
_sc_params = pltpu.CompilerParams(use_tc_tiling_on_sc=False)


@functools.partial(
    pl.kernel,
    out_type=jax.ShapeDtypeStruct((NC, NPAD), jnp.float32),
    mesh=_mesh,
    scratch_types=[
        pltpu.VMEM_SHARED((NPAD,), jnp.float32),
        pltpu.VMEM((2, CHUNK), jnp.int32),
        pltpu.VMEM((2, CHUNK), jnp.int32),
        pltpu.VMEM((CHUNK,), jnp.float32),
        pltpu.VMEM((ROWS_PT,), jnp.float32),
        pltpu.SemaphoreType.DMA,
        pltpu.SemaphoreType.DMA,
    ],
    compiler_params=_sc_params,
)
def _sc_degree(eidx_hbm, out_hbm, acc, ib0, ib1, ones, stage, si0, si1):
    c = lax.axis_index("c")
    s = lax.axis_index("s")
    w = c * NS + s
    ibs, sis = (ib0, ib1), (si0, si1)

    def _fill(i, _):
        stage[pl.ds(i * 16, 16)] = jnp.zeros((16,), jnp.float32)
        return 0
    lax.fori_loop(0, ROWS_PT // 16, _fill, 0)

    def _fill1(i, _):
        ones[pl.ds(i * 16, 16)] = jnp.ones((16,), jnp.float32)
        return 0
    lax.fori_loop(0, CHUNK // 16, _fill1, 0)

    pltpu.sync_copy(stage, acc.at[pl.ds(s * ROWS_PT, ROWS_PT)])
    plsc.subcore_barrier()

    pltpu.async_copy(eidx_hbm.at[w, 0], ib0, si0)
    pltpu.async_copy(eidx_hbm.at[w, 1], ib1, si1)

    def _pair(i, _):
        for b in (0, 1):
            k = 2 * i + b
            ib, si = ibs[b], sis[b]
            pltpu.make_async_copy(eidx_hbm.at[w, 0], ib, si).wait()
            pltpu.sync_copy(ones, acc.at[ib.at[1]], add=True)
            pltpu.async_copy(eidx_hbm.at[w, k + 2], ib, si)
        return 0
    lax.fori_loop(0, NCHUNK // 2, _pair, 0)
    pltpu.make_async_copy(eidx_hbm.at[w, 0], ib0, si0).wait()
    pltpu.make_async_copy(eidx_hbm.at[w, 0], ib1, si1).wait()

    plsc.subcore_barrier()
    pltpu.sync_copy(acc.at[pl.ds(s * ROWS_PT, ROWS_PT)], stage)
    pltpu.sync_copy(stage, out_hbm.at[c, pl.ds(s * ROWS_PT, ROWS_PT)])


@functools.partial(
    pl.kernel,
    out_type=jax.ShapeDtypeStruct((NC, NPAD, F), jnp.float32),
    mesh=_mesh,
    scratch_types=(
        [pltpu.VMEM_SHARED((NPAD, F), jnp.float32)]
        + [pltpu.VMEM((2, CHUNK), jnp.int32) for _ in range(NSLOT)]
        + [pltpu.VMEM((CHUNK, F), jnp.float32) for _ in range(NSLOT)]
        + [pltpu.SemaphoreType.DMA for _ in range(2 * NSLOT)]
    ),
    compiler_params=_sc_params,
)
def _sc_edge_pass(y_hbm, eidx_hbm, out_hbm, acc, *bufs):
    ibs = bufs[0:NSLOT]
    rs = bufs[NSLOT:2 * NSLOT]
    sis = bufs[2 * NSLOT:3 * NSLOT]
    sgs = bufs[3 * NSLOT:4 * NSLOT]
    c = lax.axis_index("c")
    s = lax.axis_index("s")
    w = c * NS + s

    def _fill(i, _):
        rs[0][i, :] = jnp.zeros((16,), jnp.float32)
        return 0
    lax.fori_loop(0, CHUNK, _fill, 0)

    def _zinit(j, _):
        pltpu.sync_copy(rs[0], acc.at[pl.ds(s * ROWS_PT + j * CHUNK, CHUNK)])
        return 0
    lax.fori_loop(0, ROWS_PT // CHUNK, _zinit, 0)
    plsc.subcore_barrier()

    for b in range(NSLOT):
        pltpu.async_copy(eidx_hbm.at[w, b], ibs[b], sis[b])
    for b in range(NSLOT - 1):
        pltpu.make_async_copy(eidx_hbm.at[w, 0], ibs[b], sis[b]).wait()
        pltpu.async_copy(y_hbm.at[ibs[b].at[0]], rs[b], sgs[b])

    def _quad(i, _):
        for b in range(NSLOT):
            k = NSLOT * i + b
            b3 = (b + NSLOT - 1) % NSLOT
            pltpu.make_async_copy(eidx_hbm.at[w, 0], ibs[b3], sis[b3]).wait()
            pltpu.async_copy(y_hbm.at[ibs[b3].at[0]], rs[b3], sgs[b3])
            pltpu.make_async_copy(y_hbm.at[ibs[b].at[0]], rs[b], sgs[b]).wait()
            pltpu.sync_copy(rs[b], acc.at[ibs[b].at[1]], add=True)
            pltpu.async_copy(eidx_hbm.at[w, k + NSLOT], ibs[b], sis[b])
        return 0
    lax.fori_loop(0, NCHUNK // NSLOT, _quad, 0)
    for b in range(NSLOT - 1):
        pltpu.make_async_copy(y_hbm.at[ibs[b].at[0]], rs[b], sgs[b]).wait()
    pltpu.make_async_copy(eidx_hbm.at[w, 0], ibs[NSLOT - 1], sis[NSLOT - 1]).wait()

    plsc.subcore_barrier()

    def _copyout(j, _):
        sl = pl.ds(s * ROWS_PT + j * CHUNK, CHUNK)
        pltpu.sync_copy(acc.at[sl], rs[0])
        pltpu.sync_copy(rs[0], out_hbm.at[c, sl])
        return 0
    lax.fori_loop(0, ROWS_PT // CHUNK, _copyout, 0)


BLK = 1024


def _tc_prep_body(x_ref, w_ref, d0_ref, d1_ref, y_ref):
    dinv = lax.rsqrt(d0_ref[...] + d1_ref[...] + 1.0)
    xw = jnp.dot(x_ref[...], w_ref[...], preferred_element_type=jnp.float32)
    y_ref[...] = xw * dinv


def _tc_mid_body(p0_ref, p1_ref, y1_ref, d0_ref, d1_ref, b_ref, w_ref, y2_ref):
    i = pl.program_id(0)
    dinv = lax.rsqrt(d0_ref[...] + d1_ref[...] + 1.0)
    h = jnp.maximum(dinv * (p0_ref[...] + p1_ref[...] + y1_ref[...]) + b_ref[...], 0.0)
    xw2 = jnp.dot(h, w_ref[...], preferred_element_type=jnp.float32)
    row = lax.broadcasted_iota(jnp.int32, (BLK, F), 0) + i * BLK
    y2_ref[...] = jnp.where(row < N, xw2 * dinv, 0.0)


FBLK = 1000


def _tc_final_body(p0_ref, p1_ref, y2_ref, d0_ref, d1_ref, b_ref, o_ref):
    dinv = lax.rsqrt(d0_ref[...] + d1_ref[...] + 1.0)
    o_ref[...] = jnp.maximum(
        dinv * (p0_ref[...] + p1_ref[...] + y2_ref[...]) + b_ref[...], 0.0)


def _row_spec(blk, width):
    return pl.BlockSpec((blk, width), lambda i: (i, 0))


def _full_spec(shape):
    return pl.BlockSpec(shape, lambda i: (0, 0))


def kernel(x, edge_index, W1, b1, W2, b2):
    ei = edge_index.astype(jnp.int32)
    pad = jnp.full((E_PAD - E,), N, jnp.int32)
    srcr = jnp.concatenate([ei[0], pad]).reshape(NW, NCHUNK, CHUNK)
    dstr = jnp.concatenate([ei[1], pad]).reshape(NW, NCHUNK, CHUNK)
    eidx = jnp.pad(jnp.stack([srcr, dstr], axis=2),
                   ((0, 0), (0, NALLOC - NCHUNK), (0, 0), (0, 0)),
                   constant_values=N)
    xp = jnp.pad(x, ((0, NPAD - N), (0, 0)))
    b1r = b1.reshape(1, F)
    b2r = b2.reshape(1, F)

    deg_parts = _sc_degree(eidx)
    d0 = deg_parts[0].reshape(NPAD, 1)
    d1 = deg_parts[1].reshape(NPAD, 1)

    y1 = pl.pallas_call(
        _tc_prep_body,
        grid=(NPAD // BLK,),
        in_specs=[_row_spec(BLK, D), _full_spec((D, F)),
                  _row_spec(BLK, 1), _row_spec(BLK, 1)],
        out_specs=_row_spec(BLK, F),
        out_shape=jax.ShapeDtypeStruct((NPAD, F), jnp.float32),
    )(xp, W1, d0, d1)

    p = _sc_edge_pass(y1, eidx)

    y2 = pl.pallas_call(
        _tc_mid_body,
        grid=(NPAD // BLK,),
        in_specs=[_row_spec(BLK, F), _row_spec(BLK, F), _row_spec(BLK, F),
                  _row_spec(BLK, 1), _row_spec(BLK, 1),
                  _full_spec((1, F)), _full_spec((F, F))],
        out_specs=_row_spec(BLK, F),
        out_shape=jax.ShapeDtypeStruct((NPAD, F), jnp.float32),
    )(p[0], p[1], y1, d0, d1, b1r, W2)

    q = _sc_edge_pass(y2, eidx)

    out = pl.pallas_call(
        _tc_final_body,
        grid=(N // FBLK,),
        in_specs=[_row_spec(FBLK, F), _row_spec(FBLK, F), _row_spec(FBLK, F),
                  _row_spec(FBLK, 1), _row_spec(FBLK, 1), _full_spec((1, F))],
        out_specs=_row_spec(FBLK, F),
        out_shape=jax.ShapeDtypeStruct((N, F), jnp.float32),
    )(q[0], q[1], y2, d0, d1, b2r)

    return out


# speedup vs baseline: 1.2043x; 1.1567x over previous
import functools

import jax
import jax.numpy as jnp
from jax import lax
from jax.experimental import pallas as pl
from jax.experimental.pallas import tpu as pltpu
from jax.experimental.pallas import tpu_sc as plsc

N = 10000
NPAD = 10240
E = 320000
D = 128
F = 16

NC = 2
NS = 16
NW = NC * NS
CHUNK = 128
NCHUNK = 80
EPT = NCHUNK * CHUNK
E_PAD = NW * EPT
NALLOC = NCHUNK + 2
ROWS_PT = NPAD // NS

_mesh = plsc.VectorSubcoreMesh(core_axis_name="c", subcore_axis_name="s")
_sc_params = pltpu.CompilerParams(use_tc_tiling_on_sc=False)


@functools.partial(
    pl.kernel,
    out_type=jax.ShapeDtypeStruct((NC, NPAD), jnp.float32),
    mesh=_mesh,
    scratch_types=[
        pltpu.VMEM_SHARED((NPAD,), jnp.float32),
        pltpu.VMEM((2, CHUNK), jnp.int32),
        pltpu.VMEM((2, CHUNK), jnp.int32),
        pltpu.VMEM((CHUNK,), jnp.float32),
        pltpu.VMEM((ROWS_PT,), jnp.float32),
        pltpu.SemaphoreType.DMA,
        pltpu.SemaphoreType.DMA,
    ],
    compiler_params=_sc_params,
)
def _sc_degree(eidx_hbm, out_hbm, acc, ib0, ib1, ones, stage, si0, si1):
    c = lax.axis_index("c")
    s = lax.axis_index("s")
    w = c * NS + s
    ibs, sis = (ib0, ib1), (si0, si1)

    def _fill(i, _):
        stage[pl.ds(i * 16, 16)] = jnp.zeros((16,), jnp.float32)
        return 0
    lax.fori_loop(0, ROWS_PT // 16, _fill, 0)

    def _fill1(i, _):
        ones[pl.ds(i * 16, 16)] = jnp.ones((16,), jnp.float32)
        return 0
    lax.fori_loop(0, CHUNK // 16, _fill1, 0)

    pltpu.sync_copy(stage, acc.at[pl.ds(s * ROWS_PT, ROWS_PT)])
    plsc.subcore_barrier()

    pltpu.async_copy(eidx_hbm.at[w, 0], ib0, si0)
    pltpu.async_copy(eidx_hbm.at[w, 1], ib1, si1)

    def _pair(i, _):
        for b in (0, 1):
            k = 2 * i + b
            ib, si = ibs[b], sis[b]
            pltpu.make_async_copy(eidx_hbm.at[w, 0], ib, si).wait()
            pltpu.sync_copy(ones, acc.at[ib.at[1]], add=True)
            pltpu.async_copy(eidx_hbm.at[w, k + 2], ib, si)
        return 0
    lax.fori_loop(0, NCHUNK // 2, _pair, 0)
    pltpu.make_async_copy(eidx_hbm.at[w, 0], ib0, si0).wait()
    pltpu.make_async_copy(eidx_hbm.at[w, 0], ib1, si1).wait()

    plsc.subcore_barrier()
    pltpu.sync_copy(acc.at[pl.ds(s * ROWS_PT, ROWS_PT)], stage)
    pltpu.sync_copy(stage, out_hbm.at[c, pl.ds(s * ROWS_PT, ROWS_PT)])


@functools.partial(
    pl.kernel,
    out_type=jax.ShapeDtypeStruct((NC, NPAD, F), jnp.float32),
    mesh=_mesh,
    scratch_types=[
        pltpu.VMEM_SHARED((NPAD, F), jnp.float32),
        pltpu.VMEM((2, CHUNK), jnp.int32),
        pltpu.VMEM((2, CHUNK), jnp.int32),
        pltpu.VMEM((CHUNK, F), jnp.float32),
        pltpu.VMEM((CHUNK, F), jnp.float32),
        pltpu.SemaphoreType.DMA,
        pltpu.SemaphoreType.DMA,
        pltpu.SemaphoreType.DMA,
        pltpu.SemaphoreType.DMA,
    ],
    compiler_params=_sc_params,
)
def _sc_edge_pass(y_hbm, eidx_hbm, out_hbm,
                  acc, ib0, ib1, r0, r1, si0, si1, sg0, sg1):
    c = lax.axis_index("c")
    s = lax.axis_index("s")
    w = c * NS + s
    ibs, rs, sis, sgs = (ib0, ib1), (r0, r1), (si0, si1), (sg0, sg1)

    def _fill(i, _):
        r0[i, :] = jnp.zeros((16,), jnp.float32)
        return 0
    lax.fori_loop(0, CHUNK, _fill, 0)

    def _zinit(j, _):
        pltpu.sync_copy(r0, acc.at[pl.ds(s * ROWS_PT + j * CHUNK, CHUNK)])
        return 0
    lax.fori_loop(0, ROWS_PT // CHUNK, _zinit, 0)
    plsc.subcore_barrier()

    pltpu.async_copy(eidx_hbm.at[w, 0], ib0, si0)
    pltpu.async_copy(eidx_hbm.at[w, 1], ib1, si1)
    pltpu.make_async_copy(eidx_hbm.at[w, 0], ib0, si0).wait()
    pltpu.async_copy(y_hbm.at[ib0.at[0]], r0, sg0)

    def _pair(i, _):
        for b in (0, 1):
            k = 2 * i + b
            b1 = 1 - b
            pltpu.make_async_copy(eidx_hbm.at[w, 0], ibs[b1], sis[b1]).wait()
            pltpu.async_copy(y_hbm.at[ibs[b1].at[0]], rs[b1], sgs[b1])
            pltpu.make_async_copy(y_hbm.at[ibs[b].at[0]], rs[b], sgs[b]).wait()
            pltpu.sync_copy(rs[b], acc.at[ibs[b].at[1]], add=True)
            pltpu.async_copy(eidx_hbm.at[w, k + 2], ibs[b], sis[b])
        return 0
    lax.fori_loop(0, NCHUNK // 2, _pair, 0)
    pltpu.make_async_copy(y_hbm.at[ib0.at[0]], r0, sg0).wait()
    pltpu.make_async_copy(eidx_hbm.at[w, 0], ib1, si1).wait()

    plsc.subcore_barrier()

    def _copyout(j, _):
        sl = pl.ds(s * ROWS_PT + j * CHUNK, CHUNK)
        pltpu.sync_copy(acc.at[sl], r0)
        pltpu.sync_copy(r0, out_hbm.at[c, sl])
        return 0
    lax.fori_loop(0, ROWS_PT // CHUNK, _copyout, 0)


BLK = 1000


def _tc_prep_body(x_ref, w_ref, d0_ref, d1_ref, y_ref):
    dinv = lax.rsqrt(d0_ref[...] + d1_ref[...] + 1.0)
    xw = jnp.dot(x_ref[...], w_ref[...], preferred_element_type=jnp.float32)
    y_ref[...] = xw * dinv


def _tc_mid_body(p0_ref, p1_ref, y1_ref, d0_ref, d1_ref, b_ref, w_ref, y2_ref):
    dinv = lax.rsqrt(d0_ref[...] + d1_ref[...] + 1.0)
    h = jnp.maximum(dinv * (p0_ref[...] + p1_ref[...] + y1_ref[...]) + b_ref[...], 0.0)
    xw2 = jnp.dot(h, w_ref[...], preferred_element_type=jnp.float32)
    y2_ref[...] = xw2 * dinv


def _tc_final_body(p0_ref, p1_ref, y2_ref, d0_ref, d1_ref, b_ref, o_ref):
    dinv = lax.rsqrt(d0_ref[...] + d1_ref[...] + 1.0)
    o_ref[...] = jnp.maximum(
        dinv * (p0_ref[...] + p1_ref[...] + y2_ref[...]) + b_ref[...], 0.0)


def _row_spec(blk, width):
    return pl.BlockSpec((blk, width), lambda i: (i, 0))


def _full_spec(shape):
    return pl.BlockSpec(shape, lambda i: (0, 0))


def kernel(x, edge_index, W1, b1, W2, b2):
    ei = edge_index.astype(jnp.int32)
    srcr = jnp.concatenate([ei[0], jnp.zeros((E_PAD - E,), jnp.int32)])
    dstr = jnp.concatenate([ei[1], jnp.full((E_PAD - E,), N, jnp.int32)])
    eidx = jnp.pad(
        jnp.stack([srcr.reshape(NW, NCHUNK, CHUNK),
                   dstr.reshape(NW, NCHUNK, CHUNK)], axis=2),
        ((0, 0), (0, NALLOC - NCHUNK), (0, 0), (0, 0)))
    b1r = b1.reshape(1, F)
    b2r = b2.reshape(1, F)

    deg_parts = _sc_degree(eidx)
    d0 = deg_parts[0].reshape(NPAD, 1)
    d1 = deg_parts[1].reshape(NPAD, 1)

    y1 = pl.pallas_call(
        _tc_prep_body,
        grid=(N // BLK,),
        in_specs=[_row_spec(BLK, D), _full_spec((D, F)),
                  _row_spec(BLK, 1), _row_spec(BLK, 1)],
        out_specs=_row_spec(BLK, F),
        out_shape=jax.ShapeDtypeStruct((N, F), jnp.float32),
    )(x, W1, d0, d1)

    p = _sc_edge_pass(y1, eidx)

    y2 = pl.pallas_call(
        _tc_mid_body,
        grid=(N // BLK,),
        in_specs=[_row_spec(BLK, F), _row_spec(BLK, F), _row_spec(BLK, F),
                  _row_spec(BLK, 1), _row_spec(BLK, 1),
                  _full_spec((1, F)), _full_spec((F, F))],
        out_specs=_row_spec(BLK, F),
        out_shape=jax.ShapeDtypeStruct((N, F), jnp.float32),
    )(p[0], p[1], y1, d0, d1, b1r, W2)

    q = _sc_edge_pass(y2, eidx)

    out = pl.pallas_call(
        _tc_final_body,
        grid=(N // BLK,),
        in_specs=[_row_spec(BLK, F), _row_spec(BLK, F), _row_spec(BLK, F),
                  _row_spec(BLK, 1), _row_spec(BLK, 1), _full_spec((1, F))],
        out_specs=_row_spec(BLK, F),
        out_shape=jax.ShapeDtypeStruct((N, F), jnp.float32),
    )(q[0], q[1], y2, d0, d1, b2r)

    return out
